# baseline (speedup 1.0000x reference)
import jax
import jax.numpy as jnp
from jax import lax
from jax.experimental import pallas as pl
from jax.experimental.pallas import tpu as pltpu
from jax.experimental.pallas import tpu_sc as plsc

B = 4
H = 512
WIDTH = 512
C = 64
HW = H * WIDTH
NPIX = B * HW
N = NPIX // 2
NCH = 3

NW = 32
PW = NPIX // NW
ROWS = PW // WIDTH
L = 16
UNR = 8

PBLK = 32768
NBT = N // PBLK
BLK = 2048
NBLK = N // BLK


def _proj_body(f_ref, w_ref, proj_ref):
    x = f_ref[...].reshape(PBLK // 2, 2 * C)
    p = lax.dot_general(w_ref[...], x, (((0,), (1,)), ((), ())),
                        preferred_element_type=jnp.float32)
    p8 = jnp.concatenate(
        [p, jnp.zeros((8 - 2 * NCH, PBLK // 2), jnp.float32)], axis=0)
    proj_ref[...] = p8[None]


def _project(xflat, w6):
    return pl.pallas_call(
        _proj_body,
        grid=(NBT,),
        in_specs=[
            pl.BlockSpec((PBLK * C,), lambda i: (i,)),
            pl.BlockSpec((2 * C, 2 * NCH), lambda i: (0, 0)),
        ],
        out_specs=pl.BlockSpec((1, 8, PBLK // 2), lambda i: (i, 0, 0)),
        out_shape=jax.ShapeDtypeStruct((NBT, 8, PBLK // 2), jnp.float32),
    )(xflat, w6)


def _sc_body(proj_hbm, idx_hbm, wb_hbm, out_hbm,
             bounds_v, idx_v, val_v, plane_v, sem):
    cid = lax.axis_index("c")
    sid = lax.axis_index("s")
    wid = sid * 2 + cid

    pltpu.sync_copy(wb_hbm.at[pl.ds(wid * L, L)], bounds_v)
    bvec = bounds_v[...]
    kstart = bvec[0]
    kcnt = bvec[1]

    lo = wid * PW
    b = wid // (NW // B)
    h0 = pl.multiple_of((wid % (NW // B)) * ROWS, ROWS)

    z16 = jnp.zeros((L,), jnp.float32)

    def _zbody(rr, _):
        for ch in range(NCH):
            for u in range(WIDTH // L):
                plane_v[ch, rr, pl.ds(u * L, L)] = z16
        return 0
    lax.fori_loop(0, ROWS, _zbody, 0)

    def _blk_body(i, _):
        k = kstart + i
        cp1 = pltpu.async_copy(idx_hbm.at[pl.ds(k * BLK, BLK)],
                               idx_v.at[pl.ds(0, BLK)], sem)
        t_off = jnp.minimum((k + 1) * BLK, N - L)
        cp2 = pltpu.async_copy(idx_hbm.at[pl.ds(t_off, L)],
                               idx_v.at[pl.ds(BLK, L)], sem)
        q0 = pl.multiple_of((k % (PBLK // BLK)) * (BLK // 2), BLK // 2)
        cp3 = pltpu.async_copy(
            proj_hbm.at[k // (PBLK // BLK), :, pl.ds(q0, BLK // 2)],
            val_v, sem)
        cp1.wait()
        cp2.wait()
        cp3.wait()

        @pl.when(k == NBLK - 1)
        def _():
            idx_v[pl.ds(BLK, L)] = jnp.full((L,), -1, jnp.int32)

        lanes = lax.iota(jnp.int32, L)
        rowp = [(lanes & 1) * NCH + ch for ch in range(NCH)]
        colp = lanes >> 1

        def _grp_body(jo, _):
            for ji in range(UNR):
                j = jo * UNR + ji
                a = idx_v[pl.ds(j * L, L)]
                nxt = idx_v[pl.ds(j * L + 1, L)]
                lid = a - lo
                m = (a != nxt) & (lid >= 0) & (lid < PW)
                lidc = jnp.clip(lid, 0, PW - 1)
                dh = lidc >> 9
                w = lidc & (WIDTH - 1)
                col = colp + j * (L // 2)
                for ch in range(NCH):
                    v = plsc.load_gather(val_v, [rowp[ch], col])
                    plsc.store_scatter(
                        plane_v, [jnp.full((L,), ch, jnp.int32), dh, w],
                        v, mask=m)
            return 0
        lax.fori_loop(0, BLK // L // UNR, _grp_body, 0)
        return 0

    lax.fori_loop(0, kcnt, _blk_body, 0)

    for ch in range(NCH):
        pltpu.sync_copy(plane_v.at[ch],
                        out_hbm.at[b, ch, pl.ds(h0, ROWS)])


_sc_scatter = pl.kernel(
    _sc_body,
    out_type=jax.ShapeDtypeStruct((B, NCH, H, WIDTH), jnp.float32),
    mesh=plsc.VectorSubcoreMesh(core_axis_name="c", subcore_axis_name="s"),
    compiler_params=pltpu.CompilerParams(needs_layout_passes=False),
    scratch_types=[
        pltpu.VMEM((L,), jnp.int32),
        pltpu.VMEM((BLK + L,), jnp.int32),
        pltpu.VMEM((8, BLK // 2), jnp.float32),
        pltpu.VMEM((NCH, ROWS, WIDTH), jnp.float32),
        pltpu.SemaphoreType.DMA,
    ],
)


def kernel(features, flat_idx, W):
    w6 = jnp.zeros((2 * C, 2 * NCH), jnp.float32)
    w6 = w6.at[:C, :NCH].set(W).at[C:, NCH:].set(W)
    proj = _project(features.reshape(N * C), w6)
    if True:
        return proj[0, :, :128]

    starts = jnp.searchsorted(flat_idx, jnp.arange(NW + 1, dtype=jnp.int32) * PW)
    st, en = starts[:-1], starts[1:]
    kstart = (st // BLK).astype(jnp.int32)
    kcnt = jnp.where(en > st, ((en - 1) // BLK).astype(jnp.int32) - kstart + 1, 0)
    wb = jnp.pad(jnp.stack([kstart, kcnt], axis=1), ((0, 0), (0, L - 2)))

    return _sc_scatter(proj, flat_idx, wb.reshape(NW * L))


# speedup vs baseline: 2.1873x; 2.1873x over previous
import jax
import jax.numpy as jnp
from jax import lax
from jax.experimental import pallas as pl
from jax.experimental.pallas import tpu as pltpu
from jax.experimental.pallas import tpu_sc as plsc

B = 4
H = 512
WIDTH = 512
C = 64
HW = H * WIDTH
NPIX = B * HW
N = NPIX // 2
NCH = 3

NW = 32
PW = NPIX // NW
ROWS = PW // WIDTH
L = 16
UNR = 8

PBLK = 16384
NBT = N // PBLK
BLK = 2048
NBLK = N // BLK


def _proj_body(ft_ref, w_ref, proj_ref):
    p = lax.dot_general(w_ref[...], ft_ref[...], (((0,), (0,)), ((), ())),
                        preferred_element_type=jnp.float32)
    p8 = jnp.concatenate([p, jnp.zeros((8 - NCH, PBLK), jnp.float32)], axis=0)
    proj_ref[...] = p8


def _project(ft, w):
    return pl.pallas_call(
        _proj_body,
        grid=(NBT,),
        in_specs=[
            pl.BlockSpec((C, PBLK), lambda i: (0, i)),
            pl.BlockSpec((C, NCH), lambda i: (0, 0)),
        ],
        out_specs=pl.BlockSpec((8, PBLK), lambda i: (0, i)),
        out_shape=jax.ShapeDtypeStruct((8, N), jnp.float32),
    )(ft, w)


def _sc_body(proj_hbm, idx_hbm, wb_hbm, out_hbm,
             bounds_v, idx_v, val_v, plane_v, sem):
    cid = lax.axis_index("c")
    sid = lax.axis_index("s")
    wid = sid * 2 + cid

    pltpu.sync_copy(wb_hbm.at[pl.ds(wid * L, L)], bounds_v)
    bvec = bounds_v[...]
    kstart = bvec[0]
    kcnt = bvec[1]

    lo = wid * PW
    b = wid // (NW // B)
    h0 = pl.multiple_of((wid % (NW // B)) * ROWS, ROWS)

    z16 = jnp.zeros((L,), jnp.float32)

    def _zbody(rr, _):
        for ch in range(NCH):
            for u in range(WIDTH // L):
                plane_v[ch, rr, pl.ds(u * L, L)] = z16
        return 0
    lax.fori_loop(0, ROWS, _zbody, 0)

    def _blk_body(i, _):
        k = kstart + i
        cp1 = pltpu.async_copy(idx_hbm.at[pl.ds(k * BLK, BLK)],
                               idx_v.at[pl.ds(0, BLK)], sem)
        t_off = jnp.minimum((k + 1) * BLK, N - L)
        cp2 = pltpu.async_copy(idx_hbm.at[pl.ds(t_off, L)],
                               idx_v.at[pl.ds(BLK, L)], sem)
        q0 = pl.multiple_of(k * BLK, BLK)
        cp3 = pltpu.async_copy(proj_hbm.at[:, pl.ds(q0, BLK)], val_v, sem)
        cp1.wait()
        cp2.wait()
        cp3.wait()

        @pl.when(k == NBLK - 1)
        def _():
            idx_v[pl.ds(BLK, L)] = jnp.full((L,), -1, jnp.int32)

        def _grp_body(jo, _):
            for ji in range(UNR):
                j = jo * UNR + ji
                a = idx_v[pl.ds(j * L, L)]
                nxt = idx_v[pl.ds(j * L + 1, L)]
                lid = a - lo
                m = (a != nxt) & (lid >= 0) & (lid < PW)
                lidc = jnp.clip(lid, 0, PW - 1)
                dh = lidc >> 9
                w = lidc & (WIDTH - 1)
                for ch in range(NCH):
                    v = val_v[ch, pl.ds(j * L, L)]
                    plsc.store_scatter(
                        plane_v, [jnp.full((L,), ch, jnp.int32), dh, w],
                        v, mask=m)
            return 0
        lax.fori_loop(0, BLK // L // UNR, _grp_body, 0)
        return 0

    lax.fori_loop(0, kcnt, _blk_body, 0)

    for ch in range(NCH):
        pltpu.sync_copy(plane_v.at[ch],
                        out_hbm.at[b, ch, pl.ds(h0, ROWS)])


_sc_scatter = pl.kernel(
    _sc_body,
    out_type=jax.ShapeDtypeStruct((B, NCH, H, WIDTH), jnp.float32),
    mesh=plsc.VectorSubcoreMesh(core_axis_name="c", subcore_axis_name="s"),
    compiler_params=pltpu.CompilerParams(needs_layout_passes=False),
    scratch_types=[
        pltpu.VMEM((L,), jnp.int32),
        pltpu.VMEM((BLK + L,), jnp.int32),
        pltpu.VMEM((8, BLK), jnp.float32),
        pltpu.VMEM((NCH, ROWS, WIDTH), jnp.float32),
        pltpu.SemaphoreType.DMA,
    ],
)


def kernel(features, flat_idx, W):
    proj = _project(features.T, W)

    starts = jnp.searchsorted(flat_idx, jnp.arange(NW + 1, dtype=jnp.int32) * PW)
    st, en = starts[:-1], starts[1:]
    kstart = (st // BLK).astype(jnp.int32)
    kcnt = jnp.where(en > st, ((en - 1) // BLK).astype(jnp.int32) - kstart + 1, 0)
    wb = jnp.pad(jnp.stack([kstart, kcnt], axis=1), ((0, 0), (0, L - 2)))

    return _sc_scatter(proj, flat_idx, wb.reshape(NW * L))
